# Initial kernel scaffold; baseline (speedup 1.0000x reference)
import jax
import jax.numpy as jnp
from jax import lax
from jax.experimental import pallas as pl
from jax.experimental.pallas import tpu as pltpu
from jax.experimental.pallas import tpu_sc as plsc

N = 10000
E = 160000
D = 256
DH = 128
DOUT = 40

NC = 2
NS = 16
DEG_W = 16

E_PER_CORE = E // NC
E_PER_TILE_DEG = E_PER_CORE // NS
DEG_B = 40
DEG_STEPS = E_PER_TILE_DEG // DEG_B
ACC_ROWS = 10240
ZCH = ACC_ROWS // NS // DEG_B

E_PER_TILE = E // NS
EB = 80
ESTEPS = E_PER_TILE // EB
ROWS_PER_TILE = N // NS

_sc_mesh = plsc.VectorSubcoreMesh(
    core_axis_name="c", subcore_axis_name="s", num_cores=NC, num_subcores=NS)


def _deg_body(src_hbm, dst_hbm, degp_hbm, acc, dst_v, fill_v):
    c = lax.axis_index("c")
    s = lax.axis_index("s")

    def fill(val):
        def row(i, _):
            fill_v[i, :] = jnp.full((DEG_W,), val, jnp.float32)
            return 0
        lax.fori_loop(0, DEG_B, row, 0)

    fill(0.0)
    def zchunk(j, _):
        pltpu.sync_copy(
            fill_v, acc.at[pl.ds(s * (ACC_ROWS // NS) + j * DEG_B, DEG_B), :])
        return 0
    lax.fori_loop(0, ZCH, zchunk, 0)
    fill(1.0)
    plsc.subcore_barrier()

    base = c * E_PER_CORE + s * E_PER_TILE_DEG
    def ebatch(i, _):
        pltpu.sync_copy(dst_hbm.at[pl.ds(base + i * DEG_B, DEG_B)], dst_v)
        pltpu.sync_copy(fill_v, acc.at[dst_v], add=True)
        return 0
    lax.fori_loop(0, DEG_STEPS, ebatch, 0)
    plsc.subcore_barrier()

    @pl.when(s == 0)
    def _():
        pltpu.sync_copy(acc.at[pl.ds(0, N), :], degp_hbm.at[c])


_deg_call = pl.kernel(
    _deg_body,
    out_type=jax.ShapeDtypeStruct((NC, N, DEG_W), jnp.float32),
    mesh=_sc_mesh,
    scratch_types=[
        pltpu.VMEM_SHARED((ACC_ROWS, DEG_W), jnp.float32),
        pltpu.VMEM((DEG_B,), jnp.int32),
        pltpu.VMEM((DEG_B, DEG_W), jnp.float32),
    ],
)


def _scat_body(z_hbm, src_hbm, dst_hbm, s_hbm, acc, src_v, dst_v, rows_v, gsem):
    c = lax.axis_index("c")
    s = lax.axis_index("s")
    r0 = s * ROWS_PER_TILE
    pltpu.sync_copy(z_hbm.at[c, pl.ds(r0, ROWS_PER_TILE), :],
                    acc.at[pl.ds(r0, ROWS_PER_TILE), :])
    plsc.subcore_barrier()

    base = s * E_PER_TILE
    def ebatch(i, _):
        eb = base + i * EB
        pltpu.sync_copy(src_hbm.at[pl.ds(eb, EB)], src_v)
        pltpu.sync_copy(dst_hbm.at[pl.ds(eb, EB)], dst_v)
        @pl.when(c == 0)
        def _():
            pltpu.async_copy(z_hbm.at[0].at[src_v], rows_v, gsem).wait()
        @pl.when(c == 1)
        def _():
            pltpu.async_copy(z_hbm.at[1].at[src_v], rows_v, gsem).wait()
        pltpu.sync_copy(rows_v, acc.at[dst_v], add=True)
        return 0
    lax.fori_loop(0, ESTEPS, ebatch, 0)
    plsc.subcore_barrier()

    pltpu.sync_copy(acc.at[pl.ds(r0, ROWS_PER_TILE), :],
                    s_hbm.at[c, pl.ds(r0, ROWS_PER_TILE), :])


_scat_call = pl.kernel(
    _scat_body,
    out_type=jax.ShapeDtypeStruct((NC, N, DH), jnp.float32),
    mesh=_sc_mesh,
    scratch_types=[
        pltpu.VMEM_SHARED((N, DH), jnp.float32),
        pltpu.VMEM((EB,), jnp.int32),
        pltpu.VMEM((EB,), jnp.int32),
        pltpu.VMEM((EB, DH), jnp.float32),
        pltpu.SemaphoreType.DMA,
    ],
)


RB = 1000
GRID = N // RB


def _dinv_of(degp_ref):
    deg = 1.0 + degp_ref[0][:, 0:1] + degp_ref[1][:, 0:1]
    return lax.rsqrt(deg)


def _tc_first_body(degp_ref, x_ref, w_ref, z_ref):
    dinv = _dinv_of(degp_ref)
    z = dinv * jnp.dot(x_ref[...], w_ref[...], preferred_element_type=jnp.float32)
    z_ref[0] = z[:, :DH]
    z_ref[1] = z[:, DH:]


def _tc_mid_body(degp_ref, s_ref, b_ref, w_ref, h_ref, zn_ref):
    dinv = _dinv_of(degp_ref)
    agg = jnp.concatenate([s_ref[0], s_ref[1]], axis=1)
    h = jnp.maximum(dinv * agg + b_ref[...][None, :], 0.0)
    h_ref[...] = h
    zn = dinv * jnp.dot(h, w_ref[...], preferred_element_type=jnp.float32)
    zn_ref[0] = zn[:, :DH]
    zn_ref[1] = zn[:, DH:]


def _tc_last_body(degp_ref, s_ref, b_ref, wc_ref, bc_ref, h_ref, y_ref):
    dinv = _dinv_of(degp_ref)
    agg = jnp.concatenate([s_ref[0], s_ref[1]], axis=1)
    h = jnp.maximum(dinv * agg + b_ref[...][None, :], 0.0)
    h_ref[...] = h
    y_ref[...] = (jnp.dot(h, wc_ref[...], preferred_element_type=jnp.float32)
                  + bc_ref[...][None, :])


_degp_spec = pl.BlockSpec((NC, RB, DEG_W), lambda i: (0, i, 0))
_half_spec = pl.BlockSpec((NC, RB, DH), lambda i: (0, i, 0))
_full_spec = pl.BlockSpec((RB, D), lambda i: (i, 0))
_w_spec = pl.BlockSpec((D, D), lambda i: (0, 0))
_b_spec = pl.BlockSpec((D,), lambda i: (0,))

_tc_first = pl.pallas_call(
    _tc_first_body,
    grid=(GRID,),
    in_specs=[_degp_spec, _full_spec, _w_spec],
    out_specs=_half_spec,
    out_shape=jax.ShapeDtypeStruct((NC, N, DH), jnp.float32),
)

_tc_mid = pl.pallas_call(
    _tc_mid_body,
    grid=(GRID,),
    in_specs=[_degp_spec, _half_spec, _b_spec, _w_spec],
    out_specs=[_full_spec, _half_spec],
    out_shape=[
        jax.ShapeDtypeStruct((N, D), jnp.float32),
        jax.ShapeDtypeStruct((NC, N, DH), jnp.float32),
    ],
)

_tc_last = pl.pallas_call(
    _tc_last_body,
    grid=(GRID,),
    in_specs=[_degp_spec, _half_spec, _b_spec,
              pl.BlockSpec((D, DOUT), lambda i: (0, 0)),
              pl.BlockSpec((DOUT,), lambda i: (0,))],
    out_specs=[_full_spec, pl.BlockSpec((RB, DOUT), lambda i: (i, 0))],
    out_shape=[
        jax.ShapeDtypeStruct((N, D), jnp.float32),
        jax.ShapeDtypeStruct((N, DOUT), jnp.float32),
    ],
)


def kernel(x, edge_index, W1, b1, W2, b2, W3, b3, Wc, bc):
    src = edge_index[0]
    dst = edge_index[1]
    degp = _deg_call(src, dst)
    z1 = _tc_first(degp, x, W1)
    s1 = _scat_call(z1, src, dst)
    h1, z2 = _tc_mid(degp, s1, b1, W2)
    s2 = _scat_call(z2, src, dst)
    h2, z3 = _tc_mid(degp, s2, b2, W3)
    s3 = _scat_call(z3, src, dst)
    h3, y = _tc_last(degp, s3, b3, Wc, bc)
    return (h1, h2, h3, y)


# speedup vs baseline: 6.9984x; 6.9984x over previous
import functools

import jax
import jax.numpy as jnp
from jax import lax
from jax.experimental import pallas as pl
from jax.experimental.pallas import tpu as pltpu
from jax.experimental.pallas import tpu_sc as plsc

N = 10000
E = 160000
D = 256
DH = 128
DOUT = 40

NC = 2
NS = 16
DEG_W = 16

E_PER_CORE = E // NC
E_PER_TILE_DEG = E_PER_CORE // NS
DEG_B = 40
DEG_STEPS = E_PER_TILE_DEG // DEG_B
ACC_ROWS = 10240
ZCH = ACC_ROWS // NS // DEG_B

E_PER_TILE = E // NS
EB = 80
ESTEPS = E_PER_TILE // EB
N_PAD = ACC_ROWS
ROWS_PER_TILE = N_PAD // NS

def _deg_body(src_hbm, dst_hbm, degp_hbm, acc, dst_v, fill_v):
    c = lax.axis_index("c")
    s = lax.axis_index("s")

    def fill(val):
        def row(i, _):
            fill_v[i, :] = jnp.full((DEG_W,), val, jnp.float32)
            return 0
        lax.fori_loop(0, DEG_B, row, 0)

    fill(0.0)
    def zchunk(j, _):
        pltpu.sync_copy(
            fill_v, acc.at[pl.ds(s * (ACC_ROWS // NS) + j * DEG_B, DEG_B), :])
        return 0
    lax.fori_loop(0, ZCH, zchunk, 0)
    fill(1.0)
    plsc.subcore_barrier()

    base = c * E_PER_CORE + s * E_PER_TILE_DEG
    def ebatch(i, _):
        pltpu.sync_copy(dst_hbm.at[pl.ds(base + i * DEG_B, DEG_B)], dst_v)
        pltpu.sync_copy(fill_v, acc.at[dst_v], add=True)
        return 0
    lax.fori_loop(0, DEG_STEPS, ebatch, 0)
    plsc.subcore_barrier()

    @pl.when(s == 0)
    def _():
        pltpu.sync_copy(acc.at[pl.ds(0, N_PAD), :], degp_hbm.at[c])


@functools.cache
def _deg_call():
    mesh = plsc.VectorSubcoreMesh(
        core_axis_name="c", subcore_axis_name="s",
        num_cores=NC, num_subcores=NS)
    return pl.kernel(
        _deg_body,
        out_type=jax.ShapeDtypeStruct((NC, N_PAD, DEG_W), jnp.float32),
        mesh=mesh,
        scratch_types=[
            pltpu.VMEM_SHARED((ACC_ROWS, DEG_W), jnp.float32),
            pltpu.VMEM((DEG_B,), jnp.int32),
            pltpu.VMEM((DEG_B, DEG_W), jnp.float32),
        ],
    )


def _scat_body(z_hbm, src_hbm, dst_hbm, s_hbm, acc, src_v, dst_v, rows_v, gsem):
    c = lax.axis_index("c")
    s = lax.axis_index("s")
    r0 = s * ROWS_PER_TILE
    pltpu.sync_copy(z_hbm.at[c, pl.ds(r0, ROWS_PER_TILE), :],
                    acc.at[pl.ds(r0, ROWS_PER_TILE), :])
    plsc.subcore_barrier()

    base = s * E_PER_TILE
    def ebatch(i, _):
        eb = base + i * EB
        pltpu.sync_copy(src_hbm.at[pl.ds(eb, EB)], src_v)
        pltpu.sync_copy(dst_hbm.at[pl.ds(eb, EB)], dst_v)
        @pl.when(c == 0)
        def _():
            pltpu.async_copy(z_hbm.at[0].at[src_v], rows_v, gsem).wait()
        @pl.when(c == 1)
        def _():
            pltpu.async_copy(z_hbm.at[1].at[src_v], rows_v, gsem).wait()
        pltpu.sync_copy(rows_v, acc.at[dst_v], add=True)
        return 0
    lax.fori_loop(0, ESTEPS, ebatch, 0)
    plsc.subcore_barrier()

    pltpu.sync_copy(acc.at[pl.ds(r0, ROWS_PER_TILE), :],
                    s_hbm.at[c, pl.ds(r0, ROWS_PER_TILE), :])


@functools.cache
def _scat_call():
    mesh = plsc.VectorSubcoreMesh(
        core_axis_name="c", subcore_axis_name="s",
        num_cores=NC, num_subcores=NS)
    return pl.kernel(
        _scat_body,
        out_type=jax.ShapeDtypeStruct((NC, N_PAD, DH), jnp.float32),
        mesh=mesh,
        scratch_types=[
            pltpu.VMEM_SHARED((N_PAD, DH), jnp.float32),
            pltpu.VMEM((EB,), jnp.int32),
            pltpu.VMEM((EB,), jnp.int32),
            pltpu.VMEM((EB, DH), jnp.float32),
            pltpu.SemaphoreType.DMA,
        ],
    )


RB = 640
GRID = N_PAD // RB


def _dinv_of(degp_ref):
    deg = 1.0 + degp_ref[0][:, 0:1] + degp_ref[1][:, 0:1]
    return lax.rsqrt(deg)


def _tc_first_body(degp_ref, x_ref, w_ref, z_ref):
    dinv = _dinv_of(degp_ref)
    z = dinv * jnp.dot(x_ref[...], w_ref[...], preferred_element_type=jnp.float32)
    z_ref[0] = z[:, :DH]
    z_ref[1] = z[:, DH:]


def _tc_mid_body(degp_ref, s_ref, b_ref, w_ref, h_ref, zn_ref):
    dinv = _dinv_of(degp_ref)
    agg = jnp.concatenate([s_ref[0], s_ref[1]], axis=1)
    h = jnp.maximum(dinv * agg + b_ref[...][None, :], 0.0)
    h_ref[...] = h
    zn = dinv * jnp.dot(h, w_ref[...], preferred_element_type=jnp.float32)
    zn_ref[0] = zn[:, :DH]
    zn_ref[1] = zn[:, DH:]


def _tc_last_body(degp_ref, s_ref, b_ref, wc_ref, bc_ref, h_ref, y_ref):
    dinv = _dinv_of(degp_ref)
    agg = jnp.concatenate([s_ref[0], s_ref[1]], axis=1)
    h = jnp.maximum(dinv * agg + b_ref[...][None, :], 0.0)
    h_ref[...] = h
    y_ref[...] = (jnp.dot(h, wc_ref[...], preferred_element_type=jnp.float32)
                  + bc_ref[...][None, :])


_degp_spec = pl.BlockSpec((NC, RB, DEG_W), lambda i: (0, i, 0))
_half_spec = pl.BlockSpec((NC, RB, DH), lambda i: (0, i, 0))
_full_spec = pl.BlockSpec((RB, D), lambda i: (i, 0))
_w_spec = pl.BlockSpec((D, D), lambda i: (0, 0))
_b_spec = pl.BlockSpec((D,), lambda i: (0,))

_tc_first = pl.pallas_call(
    _tc_first_body,
    grid=(GRID,),
    in_specs=[_degp_spec, _full_spec, _w_spec],
    out_specs=_half_spec,
    out_shape=jax.ShapeDtypeStruct((NC, N_PAD, DH), jnp.float32),
)

_tc_mid = pl.pallas_call(
    _tc_mid_body,
    grid=(GRID,),
    in_specs=[_degp_spec, _half_spec, _b_spec, _w_spec],
    out_specs=[_full_spec, _half_spec],
    out_shape=[
        jax.ShapeDtypeStruct((N, D), jnp.float32),
        jax.ShapeDtypeStruct((NC, N_PAD, DH), jnp.float32),
    ],
)

_tc_last = pl.pallas_call(
    _tc_last_body,
    grid=(GRID,),
    in_specs=[_degp_spec, _half_spec, _b_spec,
              pl.BlockSpec((D, DOUT), lambda i: (0, 0)),
              pl.BlockSpec((DOUT,), lambda i: (0,))],
    out_specs=[_full_spec, pl.BlockSpec((RB, DOUT), lambda i: (i, 0))],
    out_shape=[
        jax.ShapeDtypeStruct((N, D), jnp.float32),
        jax.ShapeDtypeStruct((N, DOUT), jnp.float32),
    ],
)


def kernel(x, edge_index, W1, b1, W2, b2, W3, b3, Wc, bc):
    src = edge_index[0]
    dst = edge_index[1]
    degp = _deg_call()(src, dst)
    z1 = _tc_first(degp, x, W1)
    s1 = _scat_call()(z1, src, dst)
    h1, z2 = _tc_mid(degp, s1, b1, W2)
    s2 = _scat_call()(z2, src, dst)
    h2, z3 = _tc_mid(degp, s2, b2, W3)
    s3 = _scat_call()(z3, src, dst)
    h3, y = _tc_last(degp, s3, b3, Wc, bc)
    return (h1, h2, h3, y)


# speedup vs baseline: 11.7404x; 1.6776x over previous
import functools

import jax
import jax.numpy as jnp
from jax import lax
from jax.experimental import pallas as pl
from jax.experimental.pallas import tpu as pltpu
from jax.experimental.pallas import tpu_sc as plsc

N = 10000
E = 160000
D = 256
DH = 128
DOUT = 40

NC = 2
NS = 16
DEG_W = 16

E_PER_CORE = E // NC
E_PER_TILE_DEG = E_PER_CORE // NS
DEG_B = 40
DEG_STEPS = E_PER_TILE_DEG // DEG_B
ACC_ROWS = 10240
ZCH = ACC_ROWS // NS // DEG_B

E_PER_TILE = E // NS
EB = 80
ESTEPS = E_PER_TILE // EB
N_PAD = ACC_ROWS
ROWS_PER_TILE = N_PAD // NS

def _deg_body(src_hbm, dst_hbm, degp_hbm, acc, dst_v, fill_v):
    c = lax.axis_index("c")
    s = lax.axis_index("s")

    def fill(val):
        def row(i, _):
            fill_v[i, :] = jnp.full((DEG_W,), val, jnp.float32)
            return 0
        lax.fori_loop(0, DEG_B, row, 0)

    fill(0.0)
    def zchunk(j, _):
        pltpu.sync_copy(
            fill_v, acc.at[pl.ds(s * (ACC_ROWS // NS) + j * DEG_B, DEG_B), :])
        return 0
    lax.fori_loop(0, ZCH, zchunk, 0)
    fill(1.0)
    plsc.subcore_barrier()

    base = c * E_PER_CORE + s * E_PER_TILE_DEG
    def ebatch(i, _):
        pltpu.sync_copy(dst_hbm.at[pl.ds(base + i * DEG_B, DEG_B)], dst_v)
        pltpu.sync_copy(fill_v, acc.at[dst_v], add=True)
        return 0
    lax.fori_loop(0, DEG_STEPS, ebatch, 0)
    plsc.subcore_barrier()

    @pl.when(s == 0)
    def _():
        pltpu.sync_copy(acc.at[pl.ds(0, N_PAD), :], degp_hbm.at[c])


@functools.cache
def _deg_call():
    mesh = plsc.VectorSubcoreMesh(
        core_axis_name="c", subcore_axis_name="s",
        num_cores=NC, num_subcores=NS)
    return pl.kernel(
        _deg_body,
        out_type=jax.ShapeDtypeStruct((NC, N_PAD, DEG_W), jnp.float32),
        mesh=mesh,
        scratch_types=[
            pltpu.VMEM_SHARED((ACC_ROWS, DEG_W), jnp.float32),
            pltpu.VMEM((DEG_B,), jnp.int32),
            pltpu.VMEM((DEG_B, DEG_W), jnp.float32),
        ],
    )


KG = 4
NGROUPS = ESTEPS // KG


def _scat_body(z_hbm, src_hbm, dst_hbm, s_hbm,
               acc, src_v, dst_v, rows_v, isem, jsem, gsem):
    c = lax.axis_index("c")
    s = lax.axis_index("s")
    r0 = s * ROWS_PER_TILE
    pltpu.sync_copy(z_hbm.at[c, pl.ds(r0, ROWS_PER_TILE), :],
                    acc.at[pl.ds(r0, ROWS_PER_TILE), :])
    plsc.subcore_barrier()

    zc = z_hbm.at[c]
    base = s * E_PER_TILE

    def batch_group(eb0, nk):
        di, dj, dg = [], [], []
        for k in range(nk):
            eb = eb0 + k * EB
            di.append(pltpu.async_copy(
                src_hbm.at[pl.ds(eb, EB)], src_v[k], isem[k]))
            dj.append(pltpu.async_copy(
                dst_hbm.at[pl.ds(eb, EB)], dst_v[k], jsem[k]))
        for k in range(nk):
            di[k].wait()
            dg.append(pltpu.async_copy(zc.at[src_v[k]], rows_v[k], gsem[k]))
        for k in range(nk):
            dg[k].wait()
            dj[k].wait()
            pltpu.sync_copy(rows_v[k], acc.at[dst_v[k]], add=True)

    def group(g, _):
        batch_group(base + g * (KG * EB), KG)
        return 0
    lax.fori_loop(0, NGROUPS, group, 0)
    batch_group(base + NGROUPS * (KG * EB), ESTEPS - NGROUPS * KG)
    plsc.subcore_barrier()

    pltpu.sync_copy(acc.at[pl.ds(r0, ROWS_PER_TILE), :],
                    s_hbm.at[c, pl.ds(r0, ROWS_PER_TILE), :])


@functools.cache
def _scat_call():
    mesh = plsc.VectorSubcoreMesh(
        core_axis_name="c", subcore_axis_name="s",
        num_cores=NC, num_subcores=NS)
    return pl.kernel(
        _scat_body,
        out_type=jax.ShapeDtypeStruct((NC, N_PAD, DH), jnp.float32),
        mesh=mesh,
        scratch_types=[
            pltpu.VMEM_SHARED((N_PAD, DH), jnp.float32),
            [pltpu.VMEM((EB,), jnp.int32)] * KG,
            [pltpu.VMEM((EB,), jnp.int32)] * KG,
            [pltpu.VMEM((EB, DH), jnp.float32)] * KG,
            [pltpu.SemaphoreType.DMA] * KG,
            [pltpu.SemaphoreType.DMA] * KG,
            [pltpu.SemaphoreType.DMA] * KG,
        ],
    )


RB = 640
GRID = N_PAD // RB


def _dinv_of(degp_ref):
    deg = 1.0 + degp_ref[0][:, 0:1] + degp_ref[1][:, 0:1]
    return lax.rsqrt(deg)


def _tc_first_body(degp_ref, x_ref, w_ref, z_ref):
    dinv = _dinv_of(degp_ref)
    z = dinv * jnp.dot(x_ref[...], w_ref[...], preferred_element_type=jnp.float32)
    z_ref[0] = z[:, :DH]
    z_ref[1] = z[:, DH:]


def _tc_mid_body(degp_ref, s_ref, b_ref, w_ref, h_ref, zn_ref):
    dinv = _dinv_of(degp_ref)
    agg = jnp.concatenate([s_ref[0], s_ref[1]], axis=1)
    h = jnp.maximum(dinv * agg + b_ref[...][None, :], 0.0)
    h_ref[...] = h
    zn = dinv * jnp.dot(h, w_ref[...], preferred_element_type=jnp.float32)
    zn_ref[0] = zn[:, :DH]
    zn_ref[1] = zn[:, DH:]


def _tc_last_body(degp_ref, s_ref, b_ref, wc_ref, bc_ref, h_ref, y_ref):
    dinv = _dinv_of(degp_ref)
    agg = jnp.concatenate([s_ref[0], s_ref[1]], axis=1)
    h = jnp.maximum(dinv * agg + b_ref[...][None, :], 0.0)
    h_ref[...] = h
    y_ref[...] = (jnp.dot(h, wc_ref[...], preferred_element_type=jnp.float32)
                  + bc_ref[...][None, :])


_degp_spec = pl.BlockSpec((NC, RB, DEG_W), lambda i: (0, i, 0))
_half_spec = pl.BlockSpec((NC, RB, DH), lambda i: (0, i, 0))
_full_spec = pl.BlockSpec((RB, D), lambda i: (i, 0))
_w_spec = pl.BlockSpec((D, D), lambda i: (0, 0))
_b_spec = pl.BlockSpec((D,), lambda i: (0,))

_tc_first = pl.pallas_call(
    _tc_first_body,
    grid=(GRID,),
    in_specs=[_degp_spec, _full_spec, _w_spec],
    out_specs=_half_spec,
    out_shape=jax.ShapeDtypeStruct((NC, N_PAD, DH), jnp.float32),
)

_tc_mid = pl.pallas_call(
    _tc_mid_body,
    grid=(GRID,),
    in_specs=[_degp_spec, _half_spec, _b_spec, _w_spec],
    out_specs=[_full_spec, _half_spec],
    out_shape=[
        jax.ShapeDtypeStruct((N, D), jnp.float32),
        jax.ShapeDtypeStruct((NC, N_PAD, DH), jnp.float32),
    ],
)

_tc_last = pl.pallas_call(
    _tc_last_body,
    grid=(GRID,),
    in_specs=[_degp_spec, _half_spec, _b_spec,
              pl.BlockSpec((D, DOUT), lambda i: (0, 0)),
              pl.BlockSpec((DOUT,), lambda i: (0,))],
    out_specs=[_full_spec, pl.BlockSpec((RB, DOUT), lambda i: (i, 0))],
    out_shape=[
        jax.ShapeDtypeStruct((N, D), jnp.float32),
        jax.ShapeDtypeStruct((N, DOUT), jnp.float32),
    ],
)


def kernel(x, edge_index, W1, b1, W2, b2, W3, b3, Wc, bc):
    src = edge_index[0]
    dst = edge_index[1]
    degp = _deg_call()(src, dst)
    z1 = _tc_first(degp, x, W1)
    s1 = _scat_call()(z1, src, dst)
    h1, z2 = _tc_mid(degp, s1, b1, W2)
    s2 = _scat_call()(z2, src, dst)
    h2, z3 = _tc_mid(degp, s2, b2, W3)
    s3 = _scat_call()(z3, src, dst)
    h3, y = _tc_last(degp, s3, b3, Wc, bc)
    return (h1, h2, h3, y)
